# Initial kernel scaffold; baseline (speedup 1.0000x reference)
import jax
import jax.numpy as jnp
from jax.experimental import pallas as pl


def kernel(query, keys, values, Wq, bq, k):
    raise NotImplementedError("write your pallas kernel here")


# speedup vs baseline: 1.4675x; 1.4675x over previous
import functools

import jax
import jax.numpy as jnp
import numpy as np
from jax import lax
from jax.experimental import pallas as pl
from jax.experimental.pallas import tpu as pltpu
from jax.experimental.pallas import tpu_sc as plsc

D = 512
NMEM = 10000
Q = 4096
KK = 32
NPAD = 10240
INV_SQRT_D = 1.0 / np.sqrt(D).astype(np.float32)
NEG = -1.0e30


QB = 256
MB = 1024
NMB = NPAD // MB


def _scores_body(q_ref, wq_ref, bq_ref, kt_ref, t_ref, c_ref, proj_ref, m_ref, l_ref):
    mb = pl.program_id(1)

    @pl.when(mb == 0)
    def _():
        proj_ref[...] = (
            jnp.dot(q_ref[...], wq_ref[...], preferred_element_type=jnp.float32)
            + bq_ref[...]
        )
        m_ref[...] = jnp.full_like(m_ref, NEG)
        l_ref[...] = jnp.zeros_like(l_ref)

    t = jnp.dot(proj_ref[...], kt_ref[...], preferred_element_type=jnp.float32)
    t = t * INV_SQRT_D
    col = mb * MB + lax.broadcasted_iota(jnp.int32, (QB, MB), 1)
    t = jnp.where(col < NMEM, t, NEG)
    t_ref[...] = t

    m_old = m_ref[...]
    m_new = jnp.maximum(m_old, jnp.max(t, axis=1, keepdims=True))
    l_ref[...] = l_ref[...] * jnp.exp(m_old - m_new) + jnp.sum(
        jnp.exp(t - m_new), axis=1, keepdims=True
    )
    m_ref[...] = m_new

    @pl.when(mb == NMB - 1)
    def _():
        c_ref[...] = m_ref[...] + jnp.log(l_ref[...])


def _scores_call(query, kt_pad, Wq, bq2):
    return pl.pallas_call(
        _scores_body,
        grid=(Q // QB, NMB),
        in_specs=[
            pl.BlockSpec((QB, D), lambda i, j: (i, 0)),
            pl.BlockSpec((D, D), lambda i, j: (0, 0)),
            pl.BlockSpec((1, D), lambda i, j: (0, 0)),
            pl.BlockSpec((D, MB), lambda i, j: (0, j)),
        ],
        out_specs=[
            pl.BlockSpec((QB, MB), lambda i, j: (i, j)),
            pl.BlockSpec((QB, 1), lambda i, j: (i, 0)),
        ],
        out_shape=[
            jax.ShapeDtypeStruct((Q, NPAD), jnp.float32),
            jax.ShapeDtypeStruct((Q, 1), jnp.float32),
        ],
        scratch_shapes=[
            pltpu.VMEM((QB, D), jnp.float32),
            pltpu.VMEM((QB, 1), jnp.float32),
            pltpu.VMEM((QB, 1), jnp.float32),
        ],
        compiler_params=pltpu.CompilerParams(
            dimension_semantics=("parallel", "arbitrary"),
        ),
    )(query, Wq, bq2, kt_pad)


NC = 2
NS = 16
L = 16
NW = NC * NS
QPW = Q // NW
NVREG = NPAD // L
CAP = 64
TRIG = 48


def _sc_body(t_hbm, c_hbm, vals_hbm, outv_hbm, outw_hbm,
             srow, cv, ci, irow, rows, wbuf, cvec, sem):
    wid = lax.axis_index("s") * NC + lax.axis_index("c")
    qbase = wid * QPW
    pltpu.sync_copy(c_hbm.at[pl.ds(qbase, QPW)], cvec)
    iota = lax.iota(jnp.int32, L)
    neg = jnp.full((L,), NEG, jnp.float32)
    zeros_i = jnp.zeros((L,), jnp.int32)

    def _compact(carry):
        cnt, tau, ak, ai, bk, bi = carry
        cnt_v = jnp.full((L,), cnt, jnp.int32)
        for g in range(CAP // L):
            valid = (iota + g * L) < cnt_v
            k_ = jnp.where(valid, cv[pl.ds(g * L, L)], neg)
            i_ = ci[pl.ds(g * L, L)]
            sk, si = plsc.sort_key_val(k_, i_)
            rk = lax.rev(sk, (0,))
            ri = lax.rev(si, (0,))
            hi = ak >= rk
            hk = jnp.where(hi, ak, rk)
            hv = jnp.where(hi, ai, ri)
            lk = jnp.where(hi, rk, ak)
            lv = jnp.where(hi, ri, ai)
            ak, ai = plsc.sort_key_val(hk, hv)
            lk, lv = plsc.sort_key_val(lk, lv)
            rlk = lax.rev(lk, (0,))
            rlv = lax.rev(lv, (0,))
            h2 = bk >= rlk
            h2k = jnp.where(h2, bk, rlk)
            h2v = jnp.where(h2, bi, rlv)
            bk, bi = plsc.sort_key_val(h2k, h2v)
        tau = jnp.full((L,), jnp.min(bk))
        return (jnp.int32(0), tau, ak, ai, bk, bi)

    def per_query(ql, carry0):
        qg = qbase + ql
        pltpu.sync_copy(t_hbm.at[qg], srow)

        def scan_body(j, carry):
            cnt, tau, ak, ai, bk, bi = carry
            v = srow[pl.ds(j * L, L)]
            m = v > tau
            n = jnp.sum(m.astype(jnp.int32))
            plsc.store_compressed(cv.at[pl.ds(cnt, L)], v, mask=m)
            plsc.store_compressed(ci.at[pl.ds(cnt, L)], iota + j * L, mask=m)
            cnt = cnt + n
            return lax.cond(cnt >= TRIG, _compact, lambda c: c,
                            (cnt, tau, ak, ai, bk, bi))

        init = (jnp.int32(0), neg, neg, zeros_i, neg, zeros_i)
        carry = lax.fori_loop(0, NVREG, scan_body, init)
        _, _, ak, ai, bk, bi = _compact(carry)

        dk0 = lax.rev(ak, (0,))
        di0 = lax.rev(ai, (0,))
        dk1 = lax.rev(bk, (0,))
        di1 = lax.rev(bi, (0,))
        cq = plsc.load_gather(cvec, [jnp.full((L,), ql, jnp.int32)])
        wbuf[pl.ds(ql * KK, L)] = jnp.exp(dk0 - cq)
        wbuf[pl.ds(ql * KK + L, L)] = jnp.exp(dk1 - cq)
        irow[pl.ds(0, L)] = di0
        irow[pl.ds(L, L)] = di1
        pltpu.async_copy(vals_hbm.at[irow], rows, sem).wait()
        pltpu.sync_copy(rows, outv_hbm.at[qg])
        return carry0

    lax.fori_loop(0, QPW, per_query, 0)
    pltpu.sync_copy(wbuf, outw_hbm.at[pl.ds(qbase * KK, QPW * KK)])


_sc_call = functools.partial(
    pl.kernel,
    mesh=plsc.VectorSubcoreMesh(core_axis_name="c", subcore_axis_name="s"),
    compiler_params=pltpu.CompilerParams(needs_layout_passes=False),
    out_type=[
        jax.ShapeDtypeStruct((Q, KK, D), jnp.float32),
        jax.ShapeDtypeStruct((Q * KK,), jnp.float32),
    ],
    scratch_types=[
        pltpu.VMEM((NPAD,), jnp.float32),
        pltpu.VMEM((CAP,), jnp.float32),
        pltpu.VMEM((CAP,), jnp.int32),
        pltpu.VMEM((KK,), jnp.int32),
        pltpu.VMEM((KK, D), jnp.float32),
        pltpu.VMEM((QPW * KK,), jnp.float32),
        pltpu.VMEM((QPW,), jnp.float32),
        pltpu.SemaphoreType.DMA,
    ],
)(_sc_body)


def kernel(query, keys, values, Wq, bq, k):
    kt_pad = jnp.pad(keys, ((0, NPAD - NMEM), (0, 0))).T
    bq2 = bq.reshape(1, D)
    t, c = _scores_call(query, kt_pad, Wq, bq2)
    outv, outw = _sc_call(t, c.reshape(Q), values)
    return outv, outw.reshape(Q, KK)


# speedup vs baseline: 2.3861x; 1.6259x over previous
import functools

import jax
import jax.numpy as jnp
import numpy as np
from jax import lax
from jax.experimental import pallas as pl
from jax.experimental.pallas import tpu as pltpu
from jax.experimental.pallas import tpu_sc as plsc

D = 512
NMEM = 10000
Q = 4096
KK = 32
NPAD = 10240
INV_SQRT_D = 1.0 / np.sqrt(D).astype(np.float32)
NEG = -1.0e30


QB = 256
MB = 1024
NMB = NPAD // MB


def _scores_body(q_ref, wq_ref, bq_ref, kt_ref, t_ref, c_ref, proj_ref, m_ref, l_ref):
    mb = pl.program_id(1)

    @pl.when(mb == 0)
    def _():
        proj_ref[...] = (
            jnp.dot(q_ref[...], wq_ref[...], preferred_element_type=jnp.float32)
            + bq_ref[...]
        )
        m_ref[...] = jnp.full_like(m_ref, NEG)
        l_ref[...] = jnp.zeros_like(l_ref)

    t = lax.dot_general(proj_ref[...], kt_ref[...], (((1,), (1,)), ((), ())),
                        preferred_element_type=jnp.float32)
    t = t * INV_SQRT_D
    col = mb * MB + lax.broadcasted_iota(jnp.int32, (QB, MB), 1)
    t = jnp.where(col < NMEM, t, NEG)
    t_ref[...] = t

    m_old = m_ref[...]
    m_new = jnp.maximum(m_old, jnp.max(t, axis=1, keepdims=True))
    l_ref[...] = l_ref[...] * jnp.exp(m_old - m_new) + jnp.sum(
        jnp.exp(t - m_new), axis=1, keepdims=True
    )
    m_ref[...] = m_new

    @pl.when(mb == NMB - 1)
    def _():
        c_ref[...] = m_ref[...] + jnp.log(l_ref[...])


def _scores_call(query, kt_pad, Wq, bq2):
    return pl.pallas_call(
        _scores_body,
        grid=(Q // QB, NMB),
        in_specs=[
            pl.BlockSpec((QB, D), lambda i, j: (i, 0)),
            pl.BlockSpec((D, D), lambda i, j: (0, 0)),
            pl.BlockSpec((1, D), lambda i, j: (0, 0)),
            pl.BlockSpec((MB, D), lambda i, j: (j, 0)),
        ],
        out_specs=[
            pl.BlockSpec((QB, MB), lambda i, j: (i, j)),
            pl.BlockSpec((QB, 1), lambda i, j: (i, 0)),
        ],
        out_shape=[
            jax.ShapeDtypeStruct((Q, NPAD), jnp.float32),
            jax.ShapeDtypeStruct((Q, 1), jnp.float32),
        ],
        scratch_shapes=[
            pltpu.VMEM((QB, D), jnp.float32),
            pltpu.VMEM((QB, 1), jnp.float32),
            pltpu.VMEM((QB, 1), jnp.float32),
        ],
        compiler_params=pltpu.CompilerParams(
            dimension_semantics=("parallel", "arbitrary"),
        ),
    )(query, Wq, bq2, kt_pad)


NC = 2
NS = 16
L = 16
NW = NC * NS
QPW = Q // NW
NVREG = NPAD // L
CAP = 64
TRIG = 48


NCH = NPAD // 16
NSTR = 16


def _sc_body(t_hbm, c_hbm, vals_hbm, outv_hbm, outw_hbm,
             srow, cm, cv, ci, irow, rows, wbuf, cvec, sem):
    wid = lax.axis_index("s") * NC + lax.axis_index("c")
    qbase = wid * QPW
    pltpu.sync_copy(c_hbm.at[pl.ds(qbase, QPW)], cvec)
    iota = lax.iota(jnp.int32, L)
    neg = jnp.full((L,), NEG, jnp.float32)
    zeros_i = jnp.zeros((L,), jnp.int32)

    def _compact(carry):
        cnt, tau, ak, ai, bk, bi = carry
        cnt_v = jnp.full((L,), cnt, jnp.int32)
        for g in range(CAP // L):
            valid = (iota + g * L) < cnt_v
            k_ = jnp.where(valid, cv[pl.ds(g * L, L)], neg)
            i_ = ci[pl.ds(g * L, L)]
            sk, si = plsc.sort_key_val(k_, i_)
            rk = lax.rev(sk, (0,))
            ri = lax.rev(si, (0,))
            hi = ak >= rk
            hk = jnp.where(hi, ak, rk)
            hv = jnp.where(hi, ai, ri)
            lk = jnp.where(hi, rk, ak)
            lv = jnp.where(hi, ri, ai)
            ak, ai = plsc.sort_key_val(hk, hv)
            lk, lv = plsc.sort_key_val(lk, lv)
            rlk = lax.rev(lk, (0,))
            rlv = lax.rev(lv, (0,))
            h2 = bk >= rlk
            h2k = jnp.where(h2, bk, rlk)
            h2v = jnp.where(h2, bi, rlv)
            bk, bi = plsc.sort_key_val(h2k, h2v)
        tau = jnp.full((L,), jnp.min(bk))
        return (jnp.int32(0), tau, ak, ai, bk, bi)

    iota640 = iota * NCH

    def per_query(ql, carry0):
        qg = qbase + ql
        pltpu.sync_copy(t_hbm.at[qg], srow)

        def cm_body(j, c0):
            acc = srow[pl.ds(j * L, L)]
            for s in range(1, NSTR):
                acc = jnp.maximum(acc, srow[pl.ds(s * NCH + j * L, L)])
            cm[pl.ds(j * L, L)] = acc
            return c0

        lax.fori_loop(0, NCH // L, cm_body, 0)

        def l1_body(jl, carry):
            cnt, tau, ak, ai, bk, bi = carry
            cmv = cm[pl.ds(jl * L, L)]
            mc = cmv > tau
            nq = jnp.sum(mc.astype(jnp.int32))
            st = (mc, cnt, tau, ak, ai, bk, bi)
            st = lax.fori_loop(0, nq, lambda t_, s: chunk_body_outer(jl, t_, s), st)
            return st[1:]

        def chunk_body_outer(jl, t_, st):
            mcur, cnt, tau, ak, ai, bk, bi = st
            f = plsc.all_reduce_ffs(mcur)
            mcur = jnp.logical_and(mcur, iota != f)
            eidx = iota640 + jl * L + f
            v = plsc.load_gather(srow, [eidx])
            m = v > tau
            n = jnp.sum(m.astype(jnp.int32))
            plsc.store_compressed(cv.at[pl.ds(cnt, L)], v, mask=m)
            plsc.store_compressed(ci.at[pl.ds(cnt, L)], eidx, mask=m)
            cnt = cnt + n
            cnt, tau, ak, ai, bk, bi = lax.cond(
                cnt >= TRIG, _compact, lambda c: c, (cnt, tau, ak, ai, bk, bi))
            return (mcur, cnt, tau, ak, ai, bk, bi)

        init = (jnp.int32(0), neg, neg, zeros_i, neg, zeros_i)
        carry = lax.fori_loop(0, NCH // L, l1_body, init)
        _, _, ak, ai, bk, bi = _compact(carry)

        dk0 = lax.rev(ak, (0,))
        di0 = lax.rev(ai, (0,))
        dk1 = lax.rev(bk, (0,))
        di1 = lax.rev(bi, (0,))
        cq = plsc.load_gather(cvec, [jnp.full((L,), ql, jnp.int32)])
        wbuf[pl.ds(ql * KK, L)] = jnp.exp(dk0 - cq)
        wbuf[pl.ds(ql * KK + L, L)] = jnp.exp(dk1 - cq)
        irow[pl.ds(0, L)] = di0
        irow[pl.ds(L, L)] = di1
        pltpu.async_copy(vals_hbm.at[irow], rows, sem).wait()
        pltpu.sync_copy(rows, outv_hbm.at[qg])
        return carry0

    lax.fori_loop(0, QPW, per_query, 0)
    pltpu.sync_copy(wbuf, outw_hbm.at[pl.ds(qbase * KK, QPW * KK)])


_sc_call = functools.partial(
    pl.kernel,
    mesh=plsc.VectorSubcoreMesh(core_axis_name="c", subcore_axis_name="s"),
    compiler_params=pltpu.CompilerParams(needs_layout_passes=False),
    out_type=[
        jax.ShapeDtypeStruct((Q, KK, D), jnp.float32),
        jax.ShapeDtypeStruct((Q * KK,), jnp.float32),
    ],
    scratch_types=[
        pltpu.VMEM((NPAD,), jnp.float32),
        pltpu.VMEM((NCH,), jnp.float32),
        pltpu.VMEM((CAP,), jnp.float32),
        pltpu.VMEM((CAP,), jnp.int32),
        pltpu.VMEM((KK,), jnp.int32),
        pltpu.VMEM((KK, D), jnp.float32),
        pltpu.VMEM((QPW * KK,), jnp.float32),
        pltpu.VMEM((QPW,), jnp.float32),
        pltpu.SemaphoreType.DMA,
    ],
)(_sc_body)


def kernel(query, keys, values, Wq, bq, k):
    kt_pad = jnp.pad(keys, ((0, NPAD - NMEM), (0, 0)))
    bq2 = bq.reshape(1, D)
    t, c = _scores_call(query, kt_pad, Wq, bq2)
    outv, outw = _sc_call(t, c.reshape(Q), values)
    return outv, outw.reshape(Q, KK)


# speedup vs baseline: 3.0457x; 1.2764x over previous
import functools

import jax
import jax.numpy as jnp
import numpy as np
from jax import lax
from jax.experimental import pallas as pl
from jax.experimental.pallas import tpu as pltpu
from jax.experimental.pallas import tpu_sc as plsc

D = 512
NMEM = 10000
Q = 4096
KK = 32
NPAD = 10240
INV_SQRT_D = 1.0 / np.sqrt(D).astype(np.float32)
NEG = -1.0e30


QB = 256
MB = 1024
NMB = NPAD // MB


def _scores_body(q_ref, wq_ref, bq_ref, kt_ref, t_ref, c_ref, proj_ref, m_ref, l_ref):
    mb = pl.program_id(1)

    @pl.when(mb == 0)
    def _():
        proj_ref[...] = (
            jnp.dot(q_ref[...], wq_ref[...], preferred_element_type=jnp.float32)
            + bq_ref[...]
        )
        m_ref[...] = jnp.full_like(m_ref, NEG)
        l_ref[...] = jnp.zeros_like(l_ref)

    t = lax.dot_general(proj_ref[...], kt_ref[...], (((1,), (1,)), ((), ())),
                        preferred_element_type=jnp.float32)
    t = t * INV_SQRT_D
    col = mb * MB + lax.broadcasted_iota(jnp.int32, (QB, MB), 1)
    t = jnp.where(col < NMEM, t, NEG)
    t_ref[...] = t

    m_old = m_ref[...]
    m_new = jnp.maximum(m_old, jnp.max(t, axis=1, keepdims=True))
    l_ref[...] = l_ref[...] * jnp.exp(m_old - m_new) + jnp.sum(
        jnp.exp(t - m_new), axis=1, keepdims=True
    )
    m_ref[...] = m_new

    @pl.when(mb == NMB - 1)
    def _():
        c_ref[...] = m_ref[...] + jnp.log(l_ref[...])


def _scores_call(query, kt_pad, Wq, bq2):
    return pl.pallas_call(
        _scores_body,
        grid=(Q // QB, NMB),
        in_specs=[
            pl.BlockSpec((QB, D), lambda i, j: (i, 0)),
            pl.BlockSpec((D, D), lambda i, j: (0, 0)),
            pl.BlockSpec((1, D), lambda i, j: (0, 0)),
            pl.BlockSpec((MB, D), lambda i, j: (j, 0)),
        ],
        out_specs=[
            pl.BlockSpec((QB, MB), lambda i, j: (i, j)),
            pl.BlockSpec((QB, 1), lambda i, j: (i, 0)),
        ],
        out_shape=[
            jax.ShapeDtypeStruct((Q, NPAD), jnp.float32),
            jax.ShapeDtypeStruct((Q, 1), jnp.float32),
        ],
        scratch_shapes=[
            pltpu.VMEM((QB, D), jnp.float32),
            pltpu.VMEM((QB, 1), jnp.float32),
            pltpu.VMEM((QB, 1), jnp.float32),
        ],
        compiler_params=pltpu.CompilerParams(
            dimension_semantics=("parallel", "arbitrary"),
        ),
    )(query, Wq, bq2, kt_pad)


NC = 2
NS = 16
L = 16
NW = NC * NS
QPW = Q // NW
NVREG = NPAD // L
CAP = 64
TRIG = 48


NCH = NPAD // 16
NSTR = 16


def _sc_body(t_hbm, c_hbm, vals_hbm, outv_hbm, outw_hbm,
             srow0, srow1, cm, cv, ci, irow0, irow1, rows0, rows1, wbuf, cvec,
             pre0, pre1, g0, g1, wb0, wb1):
    srow_b = (srow0, srow1)
    irow_b = (irow0, irow1)
    rows_b = (rows0, rows1)
    pre_s = (pre0, pre1)
    g_s = (g0, g1)
    wb_s = (wb0, wb1)
    wid = lax.axis_index("s") * NC + lax.axis_index("c")
    qbase = wid * QPW
    pltpu.sync_copy(c_hbm.at[pl.ds(qbase, QPW)], cvec)
    iota = lax.iota(jnp.int32, L)
    neg = jnp.full((L,), NEG, jnp.float32)
    zeros_i = jnp.zeros((L,), jnp.int32)

    def _compact(carry):
        cnt, tau, ak, ai, bk, bi = carry
        cnt_v = jnp.full((L,), cnt, jnp.int32)
        for g in range(CAP // L):
            valid = (iota + g * L) < cnt_v
            k_ = jnp.where(valid, cv[pl.ds(g * L, L)], neg)
            i_ = ci[pl.ds(g * L, L)]
            sk, si = plsc.sort_key_val(k_, i_)
            rk = lax.rev(sk, (0,))
            ri = lax.rev(si, (0,))
            hi = ak >= rk
            hk = jnp.where(hi, ak, rk)
            hv = jnp.where(hi, ai, ri)
            lk = jnp.where(hi, rk, ak)
            lv = jnp.where(hi, ri, ai)
            ak, ai = plsc.sort_key_val(hk, hv)
            lk, lv = plsc.sort_key_val(lk, lv)
            rlk = lax.rev(lk, (0,))
            rlv = lax.rev(lv, (0,))
            h2 = bk >= rlk
            h2k = jnp.where(h2, bk, rlk)
            h2v = jnp.where(h2, bi, rlv)
            bk, bi = plsc.sort_key_val(h2k, h2v)
        tau = jnp.full((L,), jnp.min(bk))
        return (jnp.int32(0), tau, ak, ai, bk, bi)

    iota640 = iota * NCH

    def topk_one(ql, srow):
        def cm_body(j, c0):
            acc = srow[pl.ds(j * L, L)]
            for s in range(1, NSTR):
                acc = jnp.maximum(acc, srow[pl.ds(s * NCH + j * L, L)])
            cm[pl.ds(j * L, L)] = acc
            return c0

        lax.fori_loop(0, NCH // L, cm_body, 0)

        def chunk_body_outer(jl, t_, st):
            mcur, cnt, tau, ak, ai, bk, bi = st
            f = plsc.all_reduce_ffs(mcur)
            mcur = jnp.logical_and(mcur, iota != f)
            eidx = iota640 + jl * L + f
            v = plsc.load_gather(srow, [eidx])
            m = v > tau
            n = jnp.sum(m.astype(jnp.int32))
            plsc.store_compressed(cv.at[pl.ds(cnt, L)], v, mask=m)
            plsc.store_compressed(ci.at[pl.ds(cnt, L)], eidx, mask=m)
            cnt = cnt + n
            cnt, tau, ak, ai, bk, bi = lax.cond(
                cnt >= TRIG, _compact, lambda c: c, (cnt, tau, ak, ai, bk, bi))
            return (mcur, cnt, tau, ak, ai, bk, bi)

        def l1_body(jl, carry):
            cnt, tau, ak, ai, bk, bi = carry
            cmv = cm[pl.ds(jl * L, L)]
            mc = cmv > tau
            nq = jnp.sum(mc.astype(jnp.int32))
            st = (mc, cnt, tau, ak, ai, bk, bi)
            st = lax.fori_loop(0, nq, lambda t_, s: chunk_body_outer(jl, t_, s), st)
            return st[1:]

        init = (jnp.int32(0), neg, neg, zeros_i, neg, zeros_i)
        carry = lax.fori_loop(0, NCH // L, l1_body, init)
        _, _, ak, ai, bk, bi = _compact(carry)
        return ak, ai, bk, bi

    def phase(i, p):
        ql = 2 * i + p
        qg = qbase + ql
        pltpu.make_async_copy(t_hbm.at[qg], srow_b[p], pre_s[p]).wait()
        @pl.when(ql + 1 < QPW)
        def _():
            pltpu.make_async_copy(
                t_hbm.at[qg + 1], srow_b[1 - p], pre_s[1 - p]).start()

        ak, ai, bk, bi = topk_one(ql, srow_b[p])

        dk0 = lax.rev(ak, (0,))
        di0 = lax.rev(ai, (0,))
        dk1 = lax.rev(bk, (0,))
        di1 = lax.rev(bi, (0,))
        cq = plsc.load_gather(cvec, [jnp.full((L,), ql, jnp.int32)])
        wbuf[pl.ds(ql * KK, L)] = jnp.exp(dk0 - cq)
        wbuf[pl.ds(ql * KK + L, L)] = jnp.exp(dk1 - cq)
        irow_b[p][pl.ds(0, L)] = di0
        irow_b[p][pl.ds(L, L)] = di1
        @pl.when(ql >= 2)
        def _():
            pltpu.make_async_copy(rows_b[p], outv_hbm.at[qg - 2], wb_s[p]).wait()
        pltpu.make_async_copy(vals_hbm.at[irow_b[p]], rows_b[p], g_s[p]).start()
        @pl.when(ql >= 1)
        def _():
            pltpu.make_async_copy(
                vals_hbm.at[irow_b[1 - p]], rows_b[1 - p], g_s[1 - p]).wait()
            pltpu.make_async_copy(
                rows_b[1 - p], outv_hbm.at[qg - 1], wb_s[1 - p]).start()

    def body2(i, c0):
        phase(i, 0)
        phase(i, 1)
        return c0

    pltpu.make_async_copy(t_hbm.at[qbase], srow_b[0], pre_s[0]).start()
    lax.fori_loop(0, QPW // 2, body2, 0)
    qlast = qbase + QPW - 1
    pltpu.make_async_copy(vals_hbm.at[irow_b[1]], rows_b[1], g_s[1]).wait()
    pltpu.make_async_copy(rows_b[1], outv_hbm.at[qlast], wb_s[1]).start()
    pltpu.make_async_copy(rows_b[0], outv_hbm.at[qlast - 1], wb_s[0]).wait()
    pltpu.make_async_copy(rows_b[1], outv_hbm.at[qlast], wb_s[1]).wait()
    pltpu.sync_copy(wbuf, outw_hbm.at[pl.ds(qbase * KK, QPW * KK)])


_sc_call = functools.partial(
    pl.kernel,
    mesh=plsc.VectorSubcoreMesh(core_axis_name="c", subcore_axis_name="s"),
    compiler_params=pltpu.CompilerParams(needs_layout_passes=False),
    out_type=[
        jax.ShapeDtypeStruct((Q, KK, D), jnp.float32),
        jax.ShapeDtypeStruct((Q * KK,), jnp.float32),
    ],
    scratch_types=[
        pltpu.VMEM((NPAD,), jnp.float32),
        pltpu.VMEM((NPAD,), jnp.float32),
        pltpu.VMEM((NCH,), jnp.float32),
        pltpu.VMEM((CAP,), jnp.float32),
        pltpu.VMEM((CAP,), jnp.int32),
        pltpu.VMEM((KK,), jnp.int32),
        pltpu.VMEM((KK,), jnp.int32),
        pltpu.VMEM((KK, D), jnp.float32),
        pltpu.VMEM((KK, D), jnp.float32),
        pltpu.VMEM((QPW * KK,), jnp.float32),
        pltpu.VMEM((QPW,), jnp.float32),
        pltpu.SemaphoreType.DMA,
        pltpu.SemaphoreType.DMA,
        pltpu.SemaphoreType.DMA,
        pltpu.SemaphoreType.DMA,
        pltpu.SemaphoreType.DMA,
        pltpu.SemaphoreType.DMA,
    ],
)(_sc_body)


def kernel(query, keys, values, Wq, bq, k):
    kt_pad = jnp.pad(keys, ((0, NPAD - NMEM), (0, 0)))
    bq2 = bq.reshape(1, D)
    t, c = _scores_call(query, kt_pad, Wq, bq2)
    outv, outw = _sc_call(t, c.reshape(Q), values)
    return outv, outw.reshape(Q, KK)


# speedup vs baseline: 3.1273x; 1.0268x over previous
import functools

import jax
import jax.numpy as jnp
import numpy as np
from jax import lax
from jax.experimental import pallas as pl
from jax.experimental.pallas import tpu as pltpu
from jax.experimental.pallas import tpu_sc as plsc

D = 512
NMEM = 10000
Q = 4096
KK = 32
NPAD = 10240
INV_SQRT_D = 1.0 / np.sqrt(D).astype(np.float32)
NEG = -1.0e30


QB = 256
MB = 1024
NMB = NPAD // MB


def _scores_body(q_ref, wq_ref, bq_ref, kt_ref, t_ref, c_ref, proj_ref, m_ref, l_ref):
    mb = pl.program_id(1)

    @pl.when(mb == 0)
    def _():
        proj_ref[...] = (
            jnp.dot(q_ref[...], wq_ref[...], preferred_element_type=jnp.float32)
            + bq_ref[...]
        )
        m_ref[...] = jnp.full_like(m_ref, NEG)
        l_ref[...] = jnp.zeros_like(l_ref)

    t = lax.dot_general(proj_ref[...], kt_ref[...], (((1,), (1,)), ((), ())),
                        preferred_element_type=jnp.float32)
    t = t * INV_SQRT_D
    col = mb * MB + lax.broadcasted_iota(jnp.int32, (QB, MB), 1)
    t = jnp.where(col < NMEM, t, NEG)
    t_ref[...] = t

    m_old = m_ref[...]
    m_new = jnp.maximum(m_old, jnp.max(t, axis=1, keepdims=True))
    l_ref[...] = l_ref[...] * jnp.exp(m_old - m_new) + jnp.sum(
        jnp.exp(t - m_new), axis=1, keepdims=True
    )
    m_ref[...] = m_new

    @pl.when(mb == NMB - 1)
    def _():
        c_ref[...] = m_ref[...] + jnp.log(l_ref[...])


def _scores_call(query, kt_pad, Wq, bq2):
    return pl.pallas_call(
        _scores_body,
        grid=(Q // QB, NMB),
        in_specs=[
            pl.BlockSpec((QB, D), lambda i, j: (i, 0)),
            pl.BlockSpec((D, D), lambda i, j: (0, 0)),
            pl.BlockSpec((1, D), lambda i, j: (0, 0)),
            pl.BlockSpec((MB, D), lambda i, j: (j, 0)),
        ],
        out_specs=[
            pl.BlockSpec((QB, MB), lambda i, j: (i, j)),
            pl.BlockSpec((QB, 1), lambda i, j: (i, 0)),
        ],
        out_shape=[
            jax.ShapeDtypeStruct((Q, NPAD), jnp.float32),
            jax.ShapeDtypeStruct((Q, 1), jnp.float32),
        ],
        scratch_shapes=[
            pltpu.VMEM((QB, D), jnp.float32),
            pltpu.VMEM((QB, 1), jnp.float32),
            pltpu.VMEM((QB, 1), jnp.float32),
        ],
        compiler_params=pltpu.CompilerParams(
            dimension_semantics=("parallel", "arbitrary"),
        ),
    )(query, Wq, bq2, kt_pad)


NC = 2
NS = 16
L = 16
NW = NC * NS
QPW = Q // NW
NVREG = NPAD // L
CAP = 64
TRIG = 48


NCH = NPAD // 16
NSTR = 16
PSTR = NCH + 8


def _sc_body(t_hbm, c_hbm, vals_hbm, outv_hbm, outw_hbm,
             srow0, srow1, cm, cv, ci, irow0, irow1, rows0, rows1, wbuf, cvec,
             pre0, pre1, g0, g1, wb0, wb1):
    srow_b = (srow0, srow1)
    irow_b = (irow0, irow1)
    rows_b = (rows0, rows1)
    pre_s = (pre0, pre1)
    g_s = (g0, g1)
    wb_s = (wb0, wb1)
    wid = lax.axis_index("s") * NC + lax.axis_index("c")
    qbase = wid * QPW
    pltpu.sync_copy(c_hbm.at[pl.ds(qbase, QPW)], cvec)
    iota = lax.iota(jnp.int32, L)
    neg = jnp.full((L,), NEG, jnp.float32)
    zeros_i = jnp.zeros((L,), jnp.int32)

    def _compact(carry):
        cnt, tau, ak, ai, bk, bi = carry
        cnt_v = jnp.full((L,), cnt, jnp.int32)
        for g in range(CAP // L):
            valid = (iota + g * L) < cnt_v
            k_ = jnp.where(valid, cv[pl.ds(g * L, L)], neg)
            i_ = ci[pl.ds(g * L, L)]
            sk, si = plsc.sort_key_val(k_, i_)
            rk = lax.rev(sk, (0,))
            ri = lax.rev(si, (0,))
            hi = ak >= rk
            hk = jnp.where(hi, ak, rk)
            hv = jnp.where(hi, ai, ri)
            lk = jnp.where(hi, rk, ak)
            lv = jnp.where(hi, ri, ai)
            ak, ai = plsc.sort_key_val(hk, hv)
            lk, lv = plsc.sort_key_val(lk, lv)
            rlk = lax.rev(lk, (0,))
            rlv = lax.rev(lv, (0,))
            h2 = bk >= rlk
            h2k = jnp.where(h2, bk, rlk)
            h2v = jnp.where(h2, bi, rlv)
            bk, bi = plsc.sort_key_val(h2k, h2v)
        tau = jnp.full((L,), jnp.min(bk))
        return (jnp.int32(0), tau, ak, ai, bk, bi)

    iota640 = iota * NCH
    iotap = iota * PSTR

    def topk_one(ql, srow):
        def cm_body(j, c0):
            acc = srow[pl.ds(j * L, L)]
            for s in range(1, NSTR):
                acc = jnp.maximum(acc, srow[pl.ds(s * PSTR + j * L, L)])
            cm[pl.ds(j * L, L)] = acc
            return c0

        lax.fori_loop(0, NCH // L, cm_body, 0)

        def chunk_body_outer(jl, t_, st):
            mcur, cnt, tau, ak, ai, bk, bi = st
            f = plsc.all_reduce_ffs(mcur)
            mcur = jnp.logical_and(mcur, iota != f)
            cid = jl * L + f
            v = plsc.load_gather(srow, [iotap + cid])
            m = v > tau
            n = jnp.sum(m.astype(jnp.int32))
            plsc.store_compressed(cv.at[pl.ds(cnt, L)], v, mask=m)
            plsc.store_compressed(ci.at[pl.ds(cnt, L)], iota640 + cid, mask=m)
            cnt = cnt + n
            cnt, tau, ak, ai, bk, bi = lax.cond(
                cnt >= TRIG, _compact, lambda c: c, (cnt, tau, ak, ai, bk, bi))
            return (mcur, cnt, tau, ak, ai, bk, bi)

        def l1_body(jl, carry):
            cnt, tau, ak, ai, bk, bi = carry
            cmv = cm[pl.ds(jl * L, L)]
            mc = cmv > tau
            nq = jnp.sum(mc.astype(jnp.int32))
            st = (mc, cnt, tau, ak, ai, bk, bi)
            st = lax.fori_loop(0, nq, lambda t_, s: chunk_body_outer(jl, t_, s), st)
            return st[1:]

        init = (jnp.int32(0), neg, neg, zeros_i, neg, zeros_i)
        carry = lax.fori_loop(0, NCH // L, l1_body, init)
        _, _, ak, ai, bk, bi = _compact(carry)
        return ak, ai, bk, bi

    def phase(i, p):
        ql = 2 * i + p
        qg = qbase + ql
        pltpu.make_async_copy(
            t_hbm.at[pl.ds(qg * NPAD, NPAD)],
            srow_b[p].at[pl.ds(0, NPAD)], pre_s[p]).wait()
        @pl.when(ql + 1 < QPW)
        def _():
            for s in range(NSTR):
                pltpu.make_async_copy(
                    t_hbm.at[pl.ds((qg + 1) * NPAD + s * NCH, NCH)],
                    srow_b[1 - p].at[pl.ds(s * PSTR, NCH)],
                    pre_s[1 - p]).start()

        ak, ai, bk, bi = topk_one(ql, srow_b[p])

        dk0 = lax.rev(ak, (0,))
        di0 = lax.rev(ai, (0,))
        dk1 = lax.rev(bk, (0,))
        di1 = lax.rev(bi, (0,))
        cq = plsc.load_gather(cvec, [jnp.full((L,), ql, jnp.int32)])
        wbuf[pl.ds(ql * KK, L)] = jnp.exp(dk0 - cq)
        wbuf[pl.ds(ql * KK + L, L)] = jnp.exp(dk1 - cq)
        irow_b[p][pl.ds(0, L)] = di0
        irow_b[p][pl.ds(L, L)] = di1
        @pl.when(ql >= 2)
        def _():
            pltpu.make_async_copy(rows_b[p], outv_hbm.at[qg - 2], wb_s[p]).wait()
        pltpu.make_async_copy(vals_hbm.at[irow_b[p]], rows_b[p], g_s[p]).start()
        @pl.when(ql >= 1)
        def _():
            pltpu.make_async_copy(
                vals_hbm.at[irow_b[1 - p]], rows_b[1 - p], g_s[1 - p]).wait()
            pltpu.make_async_copy(
                rows_b[1 - p], outv_hbm.at[qg - 1], wb_s[1 - p]).start()

    def body2(i, c0):
        phase(i, 0)
        phase(i, 1)
        return c0

    for s in range(NSTR):
        pltpu.make_async_copy(
            t_hbm.at[pl.ds(qbase * NPAD + s * NCH, NCH)],
            srow_b[0].at[pl.ds(s * PSTR, NCH)], pre_s[0]).start()
    lax.fori_loop(0, QPW // 2, body2, 0)
    qlast = qbase + QPW - 1
    pltpu.make_async_copy(vals_hbm.at[irow_b[1]], rows_b[1], g_s[1]).wait()
    pltpu.make_async_copy(rows_b[1], outv_hbm.at[qlast], wb_s[1]).start()
    pltpu.make_async_copy(rows_b[0], outv_hbm.at[qlast - 1], wb_s[0]).wait()
    pltpu.make_async_copy(rows_b[1], outv_hbm.at[qlast], wb_s[1]).wait()
    pltpu.sync_copy(wbuf, outw_hbm.at[pl.ds(qbase * KK, QPW * KK)])


_sc_call = functools.partial(
    pl.kernel,
    mesh=plsc.VectorSubcoreMesh(core_axis_name="c", subcore_axis_name="s"),
    compiler_params=pltpu.CompilerParams(needs_layout_passes=False),
    out_type=[
        jax.ShapeDtypeStruct((Q, KK, D), jnp.float32),
        jax.ShapeDtypeStruct((Q * KK,), jnp.float32),
    ],
    scratch_types=[
        pltpu.VMEM((NSTR * PSTR,), jnp.float32),
        pltpu.VMEM((NSTR * PSTR,), jnp.float32),
        pltpu.VMEM((NCH,), jnp.float32),
        pltpu.VMEM((CAP,), jnp.float32),
        pltpu.VMEM((CAP,), jnp.int32),
        pltpu.VMEM((KK,), jnp.int32),
        pltpu.VMEM((KK,), jnp.int32),
        pltpu.VMEM((KK, D), jnp.float32),
        pltpu.VMEM((KK, D), jnp.float32),
        pltpu.VMEM((QPW * KK,), jnp.float32),
        pltpu.VMEM((QPW,), jnp.float32),
        pltpu.SemaphoreType.DMA,
        pltpu.SemaphoreType.DMA,
        pltpu.SemaphoreType.DMA,
        pltpu.SemaphoreType.DMA,
        pltpu.SemaphoreType.DMA,
        pltpu.SemaphoreType.DMA,
    ],
)(_sc_body)


def kernel(query, keys, values, Wq, bq, k):
    kt_pad = jnp.pad(keys, ((0, NPAD - NMEM), (0, 0)))
    bq2 = bq.reshape(1, D)
    t, c = _scores_call(query, kt_pad, Wq, bq2)
    outv, outw = _sc_call(t.reshape(Q * NPAD), c.reshape(Q), values)
    return outv, outw.reshape(Q, KK)


# speedup vs baseline: 3.3442x; 1.0694x over previous
import functools

import jax
import jax.numpy as jnp
import numpy as np
from jax import lax
from jax.experimental import pallas as pl
from jax.experimental.pallas import tpu as pltpu
from jax.experimental.pallas import tpu_sc as plsc

D = 512
NMEM = 10000
Q = 4096
KK = 32
NPAD = 10240
INV_SQRT_D = 1.0 / np.sqrt(D).astype(np.float32)
NEG = -1.0e30


QB = 256
MB = 1024
NMB = NPAD // MB


def _scores_body(q_ref, wq_ref, bq_ref, kt_ref, t_ref, c_ref, proj_ref, m_ref, l_ref):
    mb = pl.program_id(1)

    @pl.when(mb == 0)
    def _():
        proj_ref[...] = (
            jnp.dot(q_ref[...], wq_ref[...], preferred_element_type=jnp.float32)
            + bq_ref[...]
        )
        m_ref[...] = jnp.full_like(m_ref, NEG)
        l_ref[...] = jnp.zeros_like(l_ref)

    t = lax.dot_general(proj_ref[...], kt_ref[...], (((1,), (1,)), ((), ())),
                        preferred_element_type=jnp.float32)
    t = t * INV_SQRT_D
    col = mb * MB + lax.broadcasted_iota(jnp.int32, (QB, MB), 1)
    t = jnp.where(col < NMEM, t, NEG)
    t_ref[...] = t

    m_old = m_ref[...]
    m_new = jnp.maximum(m_old, jnp.max(t, axis=1, keepdims=True))
    l_ref[...] = l_ref[...] * jnp.exp(m_old - m_new) + jnp.sum(
        jnp.exp(t - m_new), axis=1, keepdims=True
    )
    m_ref[...] = m_new

    @pl.when(mb == NMB - 1)
    def _():
        c_ref[...] = m_ref[...] + jnp.log(l_ref[...])


def _scores_call(query, kt_pad, Wq, bq2):
    return pl.pallas_call(
        _scores_body,
        grid=(Q // QB, NMB),
        in_specs=[
            pl.BlockSpec((QB, D), lambda i, j: (i, 0)),
            pl.BlockSpec((D, D), lambda i, j: (0, 0)),
            pl.BlockSpec((1, D), lambda i, j: (0, 0)),
            pl.BlockSpec((MB, D), lambda i, j: (j, 0)),
        ],
        out_specs=[
            pl.BlockSpec((QB, MB), lambda i, j: (i, j)),
            pl.BlockSpec((QB, 1), lambda i, j: (i, 0)),
        ],
        out_shape=[
            jax.ShapeDtypeStruct((Q, NPAD), jnp.float32),
            jax.ShapeDtypeStruct((Q, 1), jnp.float32),
        ],
        scratch_shapes=[
            pltpu.VMEM((QB, D), jnp.float32),
            pltpu.VMEM((QB, 1), jnp.float32),
            pltpu.VMEM((QB, 1), jnp.float32),
        ],
        compiler_params=pltpu.CompilerParams(
            dimension_semantics=("parallel", "arbitrary"),
        ),
    )(query, Wq, bq2, kt_pad)


NC = 2
NS = 16
L = 16
NW = NC * NS
QPW = Q // NW
NVREG = NPAD // L
CAP = 64
TRIG = 48


NCH = NPAD // 16
NSTR = 16
PSTR = NCH + 8


def _sc_body(t_hbm, c_hbm, vals_hbm, outv_hbm, outw_hbm,
             srow0, srow1, cm, cv, ci, irow0, irow1, rows0, rows1, wbuf, cvec,
             nbuf, pre0, pre1, g0, g1, wb0, wb1):
    def popcnt(mask):
        return plsc.all_reduce_population_count(mask)[0]

    srow_b = (srow0, srow1)
    irow_b = (irow0, irow1)
    rows_b = (rows0, rows1)
    pre_s = (pre0, pre1)
    g_s = (g0, g1)
    wb_s = (wb0, wb1)
    wid = lax.axis_index("s") * NC + lax.axis_index("c")
    qbase = wid * QPW
    pltpu.sync_copy(c_hbm.at[pl.ds(qbase, QPW)], cvec)
    iota = lax.iota(jnp.int32, L)
    neg = jnp.full((L,), NEG, jnp.float32)
    zeros_i = jnp.zeros((L,), jnp.int32)

    def _compact(carry):
        cnt, tau, ak, ai, bk, bi = carry
        cnt_v = jnp.full((L,), cnt, jnp.int32)
        for g in range(CAP // L):
            valid = (iota + g * L) < cnt_v
            k_ = jnp.where(valid, cv[pl.ds(g * L, L)], neg)
            i_ = ci[pl.ds(g * L, L)]
            sk, si = plsc.sort_key_val(k_, i_)
            rk = lax.rev(sk, (0,))
            ri = lax.rev(si, (0,))
            hi = ak >= rk
            hk = jnp.where(hi, ak, rk)
            hv = jnp.where(hi, ai, ri)
            lk = jnp.where(hi, rk, ak)
            lv = jnp.where(hi, ri, ai)
            ak, ai = plsc.sort_key_val(hk, hv)
            lk, lv = plsc.sort_key_val(lk, lv)
            rlk = lax.rev(lk, (0,))
            rlv = lax.rev(lv, (0,))
            h2 = bk >= rlk
            h2k = jnp.where(h2, bk, rlk)
            h2v = jnp.where(h2, bi, rlv)
            bk, bi = plsc.sort_key_val(h2k, h2v)
        tau = jnp.full((L,), jnp.min(bk))
        return (jnp.int32(0), tau, ak, ai, bk, bi)

    iota640 = iota * NCH
    iotap = iota * PSTR

    def topk_one(ql, srow):
        def cm_body(j, c0):
            acc = srow[pl.ds(j * L, L)]
            for s in range(1, NSTR):
                acc = jnp.maximum(acc, srow[pl.ds(s * PSTR + j * L, L)])
            cm[pl.ds(j * L, L)] = acc
            return c0

        lax.fori_loop(0, NCH // L, cm_body, 0)

        def chunk_body_outer(jl, t_, st):
            mcur, cnt, tau, ak, ai, bk, bi = st
            f = plsc.all_reduce_ffs(mcur)
            mcur = jnp.logical_and(mcur, iota != f)
            cid = jl * L + f
            v = plsc.load_gather(srow, [iotap + cid])
            m = v > tau
            n = popcnt(m)
            plsc.store_compressed(cv.at[pl.ds(cnt, L)], v, mask=m)
            plsc.store_compressed(ci.at[pl.ds(cnt, L)], iota640 + cid, mask=m)
            cnt = cnt + n
            cnt, tau, ak, ai, bk, bi = lax.cond(
                cnt >= TRIG, _compact, lambda c: c, (cnt, tau, ak, ai, bk, bi))
            return (mcur, cnt, tau, ak, ai, bk, bi)

        def l1_body(jl, carry):
            cnt, tau, ak, ai, bk, bi = carry
            cmv = cm[pl.ds(jl * L, L)]
            mc = cmv > tau
            nq = popcnt(mc)
            st = (mc, cnt, tau, ak, ai, bk, bi)
            st = lax.fori_loop(0, nq, lambda t_, s: chunk_body_outer(jl, t_, s), st)
            return st[1:]

        init = (jnp.int32(0), neg, neg, zeros_i, neg, zeros_i)
        carry = lax.fori_loop(0, NCH // L, l1_body, init)
        _, _, ak, ai, bk, bi = _compact(carry)
        return ak, ai, bk, bi

    def phase(i, p):
        ql = 2 * i + p
        qg = qbase + ql
        pltpu.make_async_copy(
            t_hbm.at[pl.ds(qg * NPAD, NPAD)],
            srow_b[p].at[pl.ds(0, NPAD)], pre_s[p]).wait()
        @pl.when(ql + 1 < QPW)
        def _():
            for s in range(NSTR):
                pltpu.make_async_copy(
                    t_hbm.at[pl.ds((qg + 1) * NPAD + s * NCH, NCH)],
                    srow_b[1 - p].at[pl.ds(s * PSTR, NCH)],
                    pre_s[1 - p]).start()

        ak, ai, bk, bi = topk_one(ql, srow_b[p])

        dk0 = lax.rev(ak, (0,))
        di0 = lax.rev(ai, (0,))
        dk1 = lax.rev(bk, (0,))
        di1 = lax.rev(bi, (0,))
        cq = plsc.load_gather(cvec, [jnp.full((L,), ql, jnp.int32)])
        wbuf[pl.ds(ql * KK, L)] = jnp.exp(dk0 - cq)
        wbuf[pl.ds(ql * KK + L, L)] = jnp.exp(dk1 - cq)
        irow_b[p][pl.ds(0, L)] = di0
        irow_b[p][pl.ds(L, L)] = di1
        @pl.when(ql >= 2)
        def _():
            pltpu.make_async_copy(rows_b[p], outv_hbm.at[qg - 2], wb_s[p]).wait()
        pltpu.make_async_copy(vals_hbm.at[irow_b[p]], rows_b[p], g_s[p]).start()
        @pl.when(ql >= 1)
        def _():
            pltpu.make_async_copy(
                vals_hbm.at[irow_b[1 - p]], rows_b[1 - p], g_s[1 - p]).wait()
            pltpu.make_async_copy(
                rows_b[1 - p], outv_hbm.at[qg - 1], wb_s[1 - p]).start()

    def body2(i, c0):
        phase(i, 0)
        phase(i, 1)
        return c0

    for s in range(NSTR):
        pltpu.make_async_copy(
            t_hbm.at[pl.ds(qbase * NPAD + s * NCH, NCH)],
            srow_b[0].at[pl.ds(s * PSTR, NCH)], pre_s[0]).start()
    lax.fori_loop(0, QPW // 2, body2, 0)
    qlast = qbase + QPW - 1
    pltpu.make_async_copy(vals_hbm.at[irow_b[1]], rows_b[1], g_s[1]).wait()
    pltpu.make_async_copy(rows_b[1], outv_hbm.at[qlast], wb_s[1]).start()
    pltpu.make_async_copy(rows_b[0], outv_hbm.at[qlast - 1], wb_s[0]).wait()
    pltpu.make_async_copy(rows_b[1], outv_hbm.at[qlast], wb_s[1]).wait()
    pltpu.sync_copy(wbuf, outw_hbm.at[pl.ds(qbase * KK, QPW * KK)])


_sc_call = functools.partial(
    pl.kernel,
    mesh=plsc.VectorSubcoreMesh(core_axis_name="c", subcore_axis_name="s"),
    compiler_params=pltpu.CompilerParams(needs_layout_passes=False),
    out_type=[
        jax.ShapeDtypeStruct((Q, KK, D), jnp.float32),
        jax.ShapeDtypeStruct((Q * KK,), jnp.float32),
    ],
    scratch_types=[
        pltpu.VMEM((NSTR * PSTR,), jnp.float32),
        pltpu.VMEM((NSTR * PSTR,), jnp.float32),
        pltpu.VMEM((NCH,), jnp.float32),
        pltpu.VMEM((CAP,), jnp.float32),
        pltpu.VMEM((CAP,), jnp.int32),
        pltpu.VMEM((KK,), jnp.int32),
        pltpu.VMEM((KK,), jnp.int32),
        pltpu.VMEM((KK, D), jnp.float32),
        pltpu.VMEM((KK, D), jnp.float32),
        pltpu.VMEM((QPW * KK,), jnp.float32),
        pltpu.VMEM((QPW,), jnp.float32),
        pltpu.VMEM((L,), jnp.int32),
        pltpu.SemaphoreType.DMA,
        pltpu.SemaphoreType.DMA,
        pltpu.SemaphoreType.DMA,
        pltpu.SemaphoreType.DMA,
        pltpu.SemaphoreType.DMA,
        pltpu.SemaphoreType.DMA,
    ],
)(_sc_body)


def kernel(query, keys, values, Wq, bq, k):
    kt_pad = jnp.pad(keys, ((0, NPAD - NMEM), (0, 0)))
    bq2 = bq.reshape(1, D)
    t, c = _scores_call(query, kt_pad, Wq, bq2)
    outv, outw = _sc_call(t.reshape(Q * NPAD), c.reshape(Q), values)
    return outv, outw.reshape(Q, KK)


# speedup vs baseline: 3.3542x; 1.0030x over previous
import functools

import jax
import jax.numpy as jnp
import numpy as np
from jax import lax
from jax.experimental import pallas as pl
from jax.experimental.pallas import tpu as pltpu
from jax.experimental.pallas import tpu_sc as plsc

D = 512
NMEM = 10000
Q = 4096
KK = 32
NPAD = 10240
INV_SQRT_D = 1.0 / np.sqrt(D).astype(np.float32)
NEG = -1.0e30


QB = 256
MB = 1024
NMB = NPAD // MB


def _scores_body(q_ref, wq_ref, bq_ref, kt_ref, t_ref, c_ref, proj_ref, m_ref, l_ref):
    mb = pl.program_id(1)

    @pl.when(mb == 0)
    def _():
        proj_ref[...] = (
            jnp.dot(q_ref[...], wq_ref[...], preferred_element_type=jnp.float32)
            + bq_ref[...]
        )
        m_ref[...] = jnp.full_like(m_ref, NEG)
        l_ref[...] = jnp.zeros_like(l_ref)

    t = lax.dot_general(proj_ref[...], kt_ref[...], (((1,), (1,)), ((), ())),
                        preferred_element_type=jnp.float32)
    t = t * INV_SQRT_D
    col = mb * MB + lax.broadcasted_iota(jnp.int32, (QB, MB), 1)
    t = jnp.where(col < NMEM, t, NEG)
    t_ref[...] = t

    m_old = m_ref[...]
    m_new = jnp.maximum(m_old, jnp.max(t, axis=1, keepdims=True))
    l_ref[...] = l_ref[...] * jnp.exp(m_old - m_new) + jnp.sum(
        jnp.exp(t - m_new), axis=1, keepdims=True
    )
    m_ref[...] = m_new

    @pl.when(mb == NMB - 1)
    def _():
        c_ref[...] = m_ref[...] + jnp.log(l_ref[...])


def _scores_call(query, kt_pad, Wq, bq2):
    return pl.pallas_call(
        _scores_body,
        grid=(Q // QB, NMB),
        in_specs=[
            pl.BlockSpec((QB, D), lambda i, j: (i, 0)),
            pl.BlockSpec((D, D), lambda i, j: (0, 0)),
            pl.BlockSpec((1, D), lambda i, j: (0, 0)),
            pl.BlockSpec((MB, D), lambda i, j: (j, 0)),
        ],
        out_specs=[
            pl.BlockSpec((QB, MB), lambda i, j: (i, j)),
            pl.BlockSpec((QB, 1), lambda i, j: (i, 0)),
        ],
        out_shape=[
            jax.ShapeDtypeStruct((Q, NPAD), jnp.float32),
            jax.ShapeDtypeStruct((Q, 1), jnp.float32),
        ],
        scratch_shapes=[
            pltpu.VMEM((QB, D), jnp.float32),
            pltpu.VMEM((QB, 1), jnp.float32),
            pltpu.VMEM((QB, 1), jnp.float32),
        ],
        compiler_params=pltpu.CompilerParams(
            dimension_semantics=("parallel", "arbitrary"),
        ),
    )(query, Wq, bq2, kt_pad)


NC = 2
NS = 16
L = 16
NW = NC * NS
QPW = Q // NW
CAP = 64
TRIG = 48


NCH = NPAD // 16
NSTR = 16
PSTR = NCH + 8


def _sc_body(t_hbm, c_hbm, vals_hbm, outv_hbm, outw_hbm,
             srow0, srow1, cm, cv, ci, irow0, irow1, rows0, rows1, wbuf, cvec,
             pre0, pre1, g0, g1, wb0, wb1):
    def popcnt(mask):
        return plsc.all_reduce_population_count(mask)[0]

    srow_b = (srow0, srow1)
    irow_b = (irow0, irow1)
    rows_b = (rows0, rows1)
    pre_s = (pre0, pre1)
    g_s = (g0, g1)
    wb_s = (wb0, wb1)
    wid = lax.axis_index("s") * NC + lax.axis_index("c")
    qbase = wid * QPW
    pltpu.sync_copy(c_hbm.at[pl.ds(qbase, QPW)], cvec)
    iota = lax.iota(jnp.int32, L)
    neg = jnp.full((L,), NEG, jnp.float32)
    zeros_i = jnp.zeros((L,), jnp.int32)

    def _compact(carry):
        cnt, tau, ak, ai, bk, bi = carry
        cnt_v = jnp.full((L,), cnt, jnp.int32)
        for g in range(CAP // L):
            valid = (iota + g * L) < cnt_v
            k_ = jnp.where(valid, cv[pl.ds(g * L, L)], neg)
            i_ = ci[pl.ds(g * L, L)]
            sk, si = plsc.sort_key_val(k_, i_)
            rk = lax.rev(sk, (0,))
            ri = lax.rev(si, (0,))
            hi = ak >= rk
            hk = jnp.where(hi, ak, rk)
            hv = jnp.where(hi, ai, ri)
            lk = jnp.where(hi, rk, ak)
            lv = jnp.where(hi, ri, ai)
            ak, ai = plsc.sort_key_val(hk, hv)
            lk, lv = plsc.sort_key_val(lk, lv)
            rlk = lax.rev(lk, (0,))
            rlv = lax.rev(lv, (0,))
            h2 = bk >= rlk
            h2k = jnp.where(h2, bk, rlk)
            h2v = jnp.where(h2, bi, rlv)
            bk, bi = plsc.sort_key_val(h2k, h2v)
        tau = jnp.full((L,), bk[0])
        return (jnp.int32(0), tau, ak, ai, bk, bi)

    iota640 = iota * NCH
    iotap = iota * PSTR

    def topk_one(ql, srow):
        def cm_body(j, c0):
            vs = [srow[pl.ds(s * PSTR + j * L, L)] for s in range(NSTR)]
            while len(vs) > 1:
                vs = [jnp.maximum(vs[2 * i], vs[2 * i + 1])
                      for i in range(len(vs) // 2)]
            cm[pl.ds(j * L, L)] = vs[0]
            return c0

        lax.fori_loop(0, NCH // L, cm_body, 0)

        def chunk_body_outer(jl, t_, st):
            mcur, cnt, tau, ak, ai, bk, bi = st
            f = plsc.all_reduce_ffs(mcur)
            mcur = jnp.logical_and(mcur, iota != f)
            cid = jl * L + f
            v = plsc.load_gather(srow, [iotap + cid])
            m = v > tau
            n = popcnt(m)
            plsc.store_compressed(cv.at[pl.ds(cnt, L)], v, mask=m)
            plsc.store_compressed(ci.at[pl.ds(cnt, L)], iota640 + cid, mask=m)
            cnt = cnt + n
            cnt, tau, ak, ai, bk, bi = lax.cond(
                cnt >= TRIG, _compact, lambda c: c, (cnt, tau, ak, ai, bk, bi))
            return (mcur, cnt, tau, ak, ai, bk, bi)

        def l1_body(jl, carry):
            cnt, tau, ak, ai, bk, bi = carry
            cmv = cm[pl.ds(jl * L, L)]
            mc = cmv > tau
            nq = popcnt(mc)
            st = (mc, cnt, tau, ak, ai, bk, bi)
            st = lax.fori_loop(0, nq, lambda t_, s: chunk_body_outer(jl, t_, s), st)
            return st[1:]

        init = (jnp.int32(0), neg, neg, zeros_i, neg, zeros_i)
        carry = lax.fori_loop(0, NCH // L, l1_body, init)
        _, _, ak, ai, bk, bi = _compact(carry)
        return ak, ai, bk, bi

    def phase(i, p):
        ql = 2 * i + p
        qg = qbase + ql
        pltpu.make_async_copy(
            t_hbm.at[pl.ds(qg * NPAD, NPAD)],
            srow_b[p].at[pl.ds(0, NPAD)], pre_s[p]).wait()
        @pl.when(ql + 1 < QPW)
        def _():
            for s in range(NSTR):
                pltpu.make_async_copy(
                    t_hbm.at[pl.ds((qg + 1) * NPAD + s * NCH, NCH)],
                    srow_b[1 - p].at[pl.ds(s * PSTR, NCH)],
                    pre_s[1 - p]).start()

        ak, ai, bk, bi = topk_one(ql, srow_b[p])

        dk0 = lax.rev(ak, (0,))
        di0 = lax.rev(ai, (0,))
        dk1 = lax.rev(bk, (0,))
        di1 = lax.rev(bi, (0,))
        cq = plsc.load_gather(cvec, [jnp.full((L,), ql, jnp.int32)])
        wbuf[pl.ds(ql * KK, L)] = jnp.exp(dk0 - cq)
        wbuf[pl.ds(ql * KK + L, L)] = jnp.exp(dk1 - cq)
        irow_b[p][pl.ds(0, L)] = di0
        irow_b[p][pl.ds(L, L)] = di1
        @pl.when(ql >= 2)
        def _():
            pltpu.make_async_copy(rows_b[p], outv_hbm.at[qg - 2], wb_s[p]).wait()
        pltpu.make_async_copy(vals_hbm.at[irow_b[p]], rows_b[p], g_s[p]).start()
        @pl.when(ql >= 1)
        def _():
            pltpu.make_async_copy(
                vals_hbm.at[irow_b[1 - p]], rows_b[1 - p], g_s[1 - p]).wait()
            pltpu.make_async_copy(
                rows_b[1 - p], outv_hbm.at[qg - 1], wb_s[1 - p]).start()

    def body2(i, c0):
        phase(i, 0)
        phase(i, 1)
        return c0

    for s in range(NSTR):
        pltpu.make_async_copy(
            t_hbm.at[pl.ds(qbase * NPAD + s * NCH, NCH)],
            srow_b[0].at[pl.ds(s * PSTR, NCH)], pre_s[0]).start()
    lax.fori_loop(0, QPW // 2, body2, 0)
    qlast = qbase + QPW - 1
    pltpu.make_async_copy(vals_hbm.at[irow_b[1]], rows_b[1], g_s[1]).wait()
    pltpu.make_async_copy(rows_b[1], outv_hbm.at[qlast], wb_s[1]).start()
    pltpu.make_async_copy(rows_b[0], outv_hbm.at[qlast - 1], wb_s[0]).wait()
    pltpu.make_async_copy(rows_b[1], outv_hbm.at[qlast], wb_s[1]).wait()
    pltpu.sync_copy(wbuf, outw_hbm.at[pl.ds(qbase * KK, QPW * KK)])


_sc_call = functools.partial(
    pl.kernel,
    mesh=plsc.VectorSubcoreMesh(core_axis_name="c", subcore_axis_name="s"),
    compiler_params=pltpu.CompilerParams(needs_layout_passes=False),
    out_type=[
        jax.ShapeDtypeStruct((Q, KK, D), jnp.float32),
        jax.ShapeDtypeStruct((Q * KK,), jnp.float32),
    ],
    scratch_types=[
        pltpu.VMEM((NSTR * PSTR,), jnp.float32),
        pltpu.VMEM((NSTR * PSTR,), jnp.float32),
        pltpu.VMEM((NCH,), jnp.float32),
        pltpu.VMEM((CAP,), jnp.float32),
        pltpu.VMEM((CAP,), jnp.int32),
        pltpu.VMEM((KK,), jnp.int32),
        pltpu.VMEM((KK,), jnp.int32),
        pltpu.VMEM((KK, D), jnp.float32),
        pltpu.VMEM((KK, D), jnp.float32),
        pltpu.VMEM((QPW * KK,), jnp.float32),
        pltpu.VMEM((QPW,), jnp.float32),
        pltpu.SemaphoreType.DMA,
        pltpu.SemaphoreType.DMA,
        pltpu.SemaphoreType.DMA,
        pltpu.SemaphoreType.DMA,
        pltpu.SemaphoreType.DMA,
        pltpu.SemaphoreType.DMA,
    ],
)(_sc_body)


def kernel(query, keys, values, Wq, bq, k):
    kt_pad = jnp.pad(keys, ((0, NPAD - NMEM), (0, 0)))
    bq2 = bq.reshape(1, D)
    t, c = _scores_call(query, kt_pad, Wq, bq2)
    outv, outw = _sc_call(t.reshape(Q * NPAD), c.reshape(Q), values)
    return outv, outw.reshape(Q, KK)
